# baseline (speedup 1.0000x reference)
import functools

import jax
import jax.numpy as jnp
from jax import lax
from jax.experimental import pallas as pl
from jax.experimental.pallas import tpu as pltpu
from jax.experimental.pallas import tpu_sc as plsc

N = 10000
E = 320000
D = 128

NC = 2
NS = 16
NW = NC * NS
EP = 327680
EPAD = EP - E
EW = EP // NW
CH = 80
NCHUNK = EW // CH
SB = 32
NSB = NCHUNK // SB
PAIRS = SB // 2
NP = 10240
RPS = NP // NS


def _fill_2d(ref, rows, width, value):
    def row(i, _):
        def col(j, _):
            ref[i, pl.ds(j * 16, 16)] = jnp.full((16,), value, jnp.float32)
            return 0
        lax.fori_loop(0, width // 16, col, 0)
        return 0
    lax.fori_loop(0, rows, row, 0)


def _zero_own_slice(rows_v, acc, s):
    _fill_2d(rows_v, CH, D, 0.0)
    for k in range(RPS // CH):
        pltpu.sync_copy(rows_v, acc.at[pl.ds(s * RPS + k * CH, CH)])


def _copy_out(acc, out_hbm, c, s):
    pltpu.sync_copy(acc.at[pl.ds(s * RPS, RPS)],
                    out_hbm.at[c, pl.ds(s * RPS, RPS)])


def _sc_agg_body(y_hbm, src_hbm, dst_hbm, out_hbm, src_v, dst_v,
                 rows_a, rows_b, acc, sem_ga, sem_gb, sem_sa, sem_sb):
    c = lax.axis_index("c")
    s = lax.axis_index("s")
    wid = c * NS + s

    _zero_own_slice(rows_a, acc, s)
    plsc.subcore_barrier()

    def gather(buf, sem, j):
        pltpu.async_copy(y_hbm.at[src_v.at[j]], buf, sem)

    def gather_wait(buf, sem, j):
        pltpu.make_async_copy(y_hbm.at[src_v.at[j]], buf, sem).wait()

    def scat(buf, sem, j):
        pltpu.async_copy(buf, acc.at[dst_v.at[j]], sem, add=True)

    def scat_wait(buf, sem, j):
        pltpu.make_async_copy(buf, acc.at[dst_v.at[j]], sem).wait()

    def superchunk(sb, _):
        pltpu.sync_copy(src_hbm.at[wid, sb], src_v)
        pltpu.sync_copy(dst_hbm.at[wid, sb], dst_v)
        gather(rows_a, sem_ga, 0)
        gather(rows_b, sem_gb, 1)

        def pair(p, _):
            j0 = 2 * p
            j1 = j0 + 1
            gather_wait(rows_a, sem_ga, j0)
            scat(rows_a, sem_sa, j0)
            gather_wait(rows_b, sem_gb, j1)
            scat(rows_b, sem_sb, j1)

            @pl.when(p < PAIRS - 1)
            def _():
                scat_wait(rows_a, sem_sa, j0)
                gather(rows_a, sem_ga, j0 + 2)
                scat_wait(rows_b, sem_sb, j1)
                gather(rows_b, sem_gb, j1 + 2)
            return 0
        lax.fori_loop(0, PAIRS, pair, 0)
        scat_wait(rows_a, sem_sa, SB - 2)
        scat_wait(rows_b, sem_sb, SB - 1)
        return 0
    lax.fori_loop(0, NSB, superchunk, 0)

    plsc.subcore_barrier()
    _copy_out(acc, out_hbm, c, s)


def _sc_cnt_body(dst_hbm, out_hbm, dst_v, rows_v, acc, sem):
    c = lax.axis_index("c")
    s = lax.axis_index("s")
    wid = c * NS + s

    _zero_own_slice(rows_v, acc, s)
    plsc.subcore_barrier()

    _fill_2d(rows_v, CH, D, 1.0)

    def superchunk(sb, _):
        pltpu.sync_copy(dst_hbm.at[wid, sb], dst_v)

        def group(g, _):
            for t in range(8):
                pltpu.async_copy(rows_v, acc.at[dst_v.at[g * 8 + t]], sem,
                                 add=True)
            for t in range(8):
                pltpu.make_async_copy(rows_v, acc.at[dst_v.at[g * 8 + t]],
                                      sem).wait()
            return 0
        lax.fori_loop(0, SB // 8, group, 0)
        return 0
    lax.fori_loop(0, NSB, superchunk, 0)

    plsc.subcore_barrier()
    _copy_out(acc, out_hbm, c, s)


@functools.lru_cache(maxsize=None)
def _make_sc_agg():
    mesh = plsc.VectorSubcoreMesh(core_axis_name="c", subcore_axis_name="s",
                                  num_cores=NC, num_subcores=NS)
    return pl.kernel(
        _sc_agg_body,
        out_type=[jax.ShapeDtypeStruct((NC, NP, D), jnp.float32)],
        mesh=mesh,
        scratch_types=[
            pltpu.VMEM((SB, CH), jnp.int32),
            pltpu.VMEM((SB, CH), jnp.int32),
            pltpu.VMEM((CH, D), jnp.float32),
            pltpu.VMEM((CH, D), jnp.float32),
            pltpu.VMEM_SHARED((NP, D), jnp.float32),
            pltpu.SemaphoreType.DMA,
            pltpu.SemaphoreType.DMA,
            pltpu.SemaphoreType.DMA,
            pltpu.SemaphoreType.DMA,
        ],
    )


@functools.lru_cache(maxsize=None)
def _make_sc_cnt():
    mesh = plsc.VectorSubcoreMesh(core_axis_name="c", subcore_axis_name="s",
                                  num_cores=NC, num_subcores=NS)
    return pl.kernel(
        _sc_cnt_body,
        out_type=[jax.ShapeDtypeStruct((NC, NP, D), jnp.float32)],
        mesh=mesh,
        scratch_types=[
            pltpu.VMEM((SB, CH), jnp.int32),
            pltpu.VMEM((CH, D), jnp.float32),
            pltpu.VMEM_SHARED((NP, D), jnp.float32),
            pltpu.SemaphoreType.DMA,
        ],
    )


RB = 1000


def _tc_combine_body(relu, acc_ref, cnt_ref, h_ref, wl_ref, b_ref, wr_ref,
                     out_ref):
    a = acc_ref[0] + acc_ref[1]
    cnt = cnt_ref[0, :, 0:1] + cnt_ref[1, :, 0:1]
    inv = 1.0 / jnp.maximum(cnt, 1.0)
    m = a * inv
    out = (jnp.dot(m, wl_ref[...], preferred_element_type=jnp.float32)
           + b_ref[...]
           + jnp.dot(h_ref[...], wr_ref[...],
                     preferred_element_type=jnp.float32))
    if relu:
        out = jnp.maximum(out, 0.0)
    out_ref[...] = out


@functools.lru_cache(maxsize=None)
def _make_combine(relu):
    return pl.pallas_call(
        functools.partial(_tc_combine_body, relu),
        grid=(N // RB,),
        in_specs=[
            pl.BlockSpec((NC, RB, D), lambda i: (0, i, 0)),
            pl.BlockSpec((NC, RB, D), lambda i: (0, i, 0)),
            pl.BlockSpec((RB, D), lambda i: (i, 0)),
            pl.BlockSpec((D, D), lambda i: (0, 0)),
            pl.BlockSpec((1, D), lambda i: (0, 0)),
            pl.BlockSpec((D, D), lambda i: (0, 0)),
        ],
        out_specs=pl.BlockSpec((RB, D), lambda i: (i, 0)),
        out_shape=jax.ShapeDtypeStruct((N, D), jnp.float32),
    )


def kernel(x, edge_index, W_l0, b_l0, W_r0, W_l1, b_l1, W_r1, W_l2, b_l2,
           W_r2):
    pad_src = jnp.zeros((EPAD,), jnp.int32)
    pad_dst = N + (jnp.arange(EPAD, dtype=jnp.int32) % (NP - N))
    src = jnp.concatenate([edge_index[0], pad_src]).reshape(NW, NSB, SB, CH)
    dst = jnp.concatenate([edge_index[1], pad_dst]).reshape(NW, NSB, SB, CH)
    sc_agg, sc_cnt = _make_sc_agg(), _make_sc_cnt()
    combine_relu, combine_last = _make_combine(True), _make_combine(False)

    (cnt2,) = sc_cnt(dst)
    (acc2,) = sc_agg(x, src, dst)
    h1 = combine_relu(acc2, cnt2, x, W_l0, b_l0.reshape(1, D), W_r0)
    (acc2,) = sc_agg(h1, src, dst)
    h2 = combine_relu(acc2, cnt2, h1, W_l1, b_l1.reshape(1, D), W_r1)
    (acc2,) = sc_agg(h2, src, dst)
    return combine_last(acc2, cnt2, h2, W_l2, b_l2.reshape(1, D), W_r2)


# speedup vs baseline: 2.4854x; 2.4854x over previous
import functools

import jax
import jax.numpy as jnp
from jax import lax
from jax.experimental import pallas as pl
from jax.experimental.pallas import tpu as pltpu
from jax.experimental.pallas import tpu_sc as plsc

N = 10000
E = 320000
D = 128

NC = 2
NS = 16
NW = NC * NS
EP = 327680
EPAD = EP - E
EW = EP // NW
CH = 80
NCHUNK = EW // CH
SB = 32
NSB = NCHUNK // SB
PAIRS = SB // 2
NP = 10240
RPS = NP // NS


def _fill_2d(ref, rows, width, value):
    def row(i, _):
        def col(j, _):
            ref[i, pl.ds(j * 16, 16)] = jnp.full((16,), value, jnp.float32)
            return 0
        lax.fori_loop(0, width // 16, col, 0)
        return 0
    lax.fori_loop(0, rows, row, 0)


def _zero_own_slice(rows_v, acc, s):
    _fill_2d(rows_v, CH, D, 0.0)
    for k in range(RPS // CH):
        pltpu.sync_copy(rows_v, acc.at[pl.ds(s * RPS + k * CH, CH)])


def _copy_out(acc, out_hbm, c, s):
    pltpu.sync_copy(acc.at[pl.ds(s * RPS, RPS)],
                    out_hbm.at[c, pl.ds(s * RPS, RPS)])


def _sc_agg_body(y_hbm, src_hbm, dst_hbm, out_hbm, src_v, dst_v,
                 rows_a, rows_b, acc, sem_ga, sem_gb, sem_sa, sem_sb):
    c = lax.axis_index("c")
    s = lax.axis_index("s")
    wid = c * NS + s

    _zero_own_slice(rows_a, acc, s)
    plsc.subcore_barrier()

    def gather(buf, sem, j):
        pltpu.async_copy(y_hbm.at[src_v.at[j]], buf, sem)

    def gather_wait(buf, sem, j):
        pltpu.make_async_copy(y_hbm.at[src_v.at[j]], buf, sem).wait()

    def scat(buf, sem, j):
        pltpu.async_copy(buf, acc.at[dst_v.at[j]], sem, add=True)

    def scat_wait(buf, sem, j):
        pltpu.make_async_copy(buf, acc.at[dst_v.at[j]], sem).wait()

    def superchunk(sb, _):
        pltpu.sync_copy(src_hbm.at[wid, sb], src_v)
        pltpu.sync_copy(dst_hbm.at[wid, sb], dst_v)
        gather(rows_a, sem_ga, 0)
        gather(rows_b, sem_gb, 1)

        def pair(p, _):
            j0 = 2 * p
            j1 = j0 + 1
            gather_wait(rows_a, sem_ga, j0)
            scat(rows_a, sem_sa, j0)
            gather_wait(rows_b, sem_gb, j1)
            scat(rows_b, sem_sb, j1)

            @pl.when(p < PAIRS - 1)
            def _():
                scat_wait(rows_a, sem_sa, j0)
                gather(rows_a, sem_ga, j0 + 2)
                scat_wait(rows_b, sem_sb, j1)
                gather(rows_b, sem_gb, j1 + 2)
            return 0
        lax.fori_loop(0, PAIRS, pair, 0)
        scat_wait(rows_a, sem_sa, SB - 2)
        scat_wait(rows_b, sem_sb, SB - 1)
        return 0
    lax.fori_loop(0, NSB, superchunk, 0)

    plsc.subcore_barrier()
    _copy_out(acc, out_hbm, c, s)


def _sc_cnt_body(dst_hbm, out_hbm, dst_v, rows_v, acc, sem):
    c = lax.axis_index("c")
    s = lax.axis_index("s")
    wid = c * NS + s

    _zero_own_slice(rows_v, acc, s)
    plsc.subcore_barrier()

    _fill_2d(rows_v, CH, D, 1.0)

    def superchunk(sb, _):
        pltpu.sync_copy(dst_hbm.at[wid, sb], dst_v)

        def group(g, _):
            for t in range(8):
                pltpu.async_copy(rows_v, acc.at[dst_v.at[g * 8 + t]], sem,
                                 add=True)
            for t in range(8):
                pltpu.make_async_copy(rows_v, acc.at[dst_v.at[g * 8 + t]],
                                      sem).wait()
            return 0
        lax.fori_loop(0, SB // 8, group, 0)
        return 0
    lax.fori_loop(0, NSB, superchunk, 0)

    plsc.subcore_barrier()
    _copy_out(acc, out_hbm, c, s)


@functools.lru_cache(maxsize=None)
def _make_sc_agg():
    mesh = plsc.VectorSubcoreMesh(core_axis_name="c", subcore_axis_name="s",
                                  num_cores=NC, num_subcores=NS)
    return pl.kernel(
        _sc_agg_body,
        out_type=[jax.ShapeDtypeStruct((NC, NP, D), jnp.float32)],
        mesh=mesh,
        scratch_types=[
            pltpu.VMEM((SB, CH), jnp.int32),
            pltpu.VMEM((SB, CH), jnp.int32),
            pltpu.VMEM((CH, D), jnp.float32),
            pltpu.VMEM((CH, D), jnp.float32),
            pltpu.VMEM_SHARED((NP, D), jnp.float32),
            pltpu.SemaphoreType.DMA,
            pltpu.SemaphoreType.DMA,
            pltpu.SemaphoreType.DMA,
            pltpu.SemaphoreType.DMA,
        ],
    )


@functools.lru_cache(maxsize=None)
def _make_sc_cnt():
    mesh = plsc.VectorSubcoreMesh(core_axis_name="c", subcore_axis_name="s",
                                  num_cores=NC, num_subcores=NS)
    return pl.kernel(
        _sc_cnt_body,
        out_type=[jax.ShapeDtypeStruct((NC, NP, D), jnp.float32)],
        mesh=mesh,
        scratch_types=[
            pltpu.VMEM((SB, CH), jnp.int32),
            pltpu.VMEM((CH, D), jnp.float32),
            pltpu.VMEM_SHARED((NP, D), jnp.float32),
            pltpu.SemaphoreType.DMA,
        ],
    )


def _tc_combine_body(relu, padded, rb, acc_ref, cnt_ref, h_ref, wl_ref,
                     b_ref, wr_ref, out_ref):
    a = acc_ref[0] + acc_ref[1]
    cnt = cnt_ref[0, :, 0:1] + cnt_ref[1, :, 0:1]
    inv = 1.0 / jnp.maximum(cnt, 1.0)
    m = a * inv
    out = (jnp.dot(m, wl_ref[...], preferred_element_type=jnp.float32)
           + b_ref[...]
           + jnp.dot(h_ref[...], wr_ref[...],
                     preferred_element_type=jnp.float32))
    if relu:
        out = jnp.maximum(out, 0.0)
    if padded:
        rid = (jax.lax.broadcasted_iota(jnp.int32, (rb, 1), 0)
               + pl.program_id(0) * rb)
        out = jnp.where(rid < N, out, 0.0)
    out_ref[...] = out


@functools.lru_cache(maxsize=None)
def _make_combine(relu, padded):
    rb = 1024 if padded else 1000
    n_out = NP if padded else N
    return pl.pallas_call(
        functools.partial(_tc_combine_body, relu, padded, rb),
        grid=(n_out // rb,),
        in_specs=[
            pl.BlockSpec((NC, rb, D), lambda i: (0, i, 0)),
            pl.BlockSpec((NC, rb, D), lambda i: (0, i, 0)),
            pl.BlockSpec((rb, D), lambda i: (i, 0)),
            pl.BlockSpec((D, D), lambda i: (0, 0)),
            pl.BlockSpec((1, D), lambda i: (0, 0)),
            pl.BlockSpec((D, D), lambda i: (0, 0)),
        ],
        out_specs=pl.BlockSpec((rb, D), lambda i: (i, 0)),
        out_shape=jax.ShapeDtypeStruct((n_out, D), jnp.float32),
    )


def kernel(x, edge_index, W_l0, b_l0, W_r0, W_l1, b_l1, W_r1, W_l2, b_l2,
           W_r2):
    PW = EPAD // NW
    k = jnp.arange(PW, dtype=jnp.int32)[None, :]
    w = jnp.arange(NW, dtype=jnp.int32)[:, None]
    pad_src = jnp.broadcast_to(N + k % (NP - N), (NW, PW))
    pad_dst_agg = (w * 313 + k * 41) % N
    pad_dst_cnt = jnp.broadcast_to(N + k % (NP - N), (NW, PW))
    r_src = edge_index[0].reshape(NW, E // NW)
    r_dst = edge_index[1].reshape(NW, E // NW)
    shp = (NW, NSB, SB, CH)
    src = jnp.concatenate([r_src, pad_src], axis=1).reshape(shp)
    dst = jnp.concatenate([r_dst, pad_dst_agg], axis=1).reshape(shp)
    dst_c = jnp.concatenate([r_dst, pad_dst_cnt], axis=1).reshape(shp)
    x_p = jnp.concatenate([x, jnp.zeros((NP - N, D), jnp.float32)])

    sc_agg, sc_cnt = _make_sc_agg(), _make_sc_cnt()
    combine_mid = _make_combine(True, True)
    combine_last = _make_combine(False, False)

    (cnt2,) = sc_cnt(dst_c)
    (acc2,) = sc_agg(x_p, src, dst)
    h1 = combine_mid(acc2, cnt2, x_p, W_l0, b_l0.reshape(1, D), W_r0)
    (acc2,) = sc_agg(h1, src, dst)
    h2 = combine_mid(acc2, cnt2, h1, W_l1, b_l1.reshape(1, D), W_r1)
    (acc2,) = sc_agg(h2, src, dst)
    return combine_last(acc2, cnt2, h2, W_l2, b_l2.reshape(1, D), W_r2)


# speedup vs baseline: 2.5494x; 1.0257x over previous
import functools

import jax
import jax.numpy as jnp
from jax import lax
from jax.experimental import pallas as pl
from jax.experimental.pallas import tpu as pltpu
from jax.experimental.pallas import tpu_sc as plsc

N = 10000
E = 320000
D = 128

NC = 2
NS = 16
NW = NC * NS
EP = 327680
EPAD = EP - E
EW = EP // NW
CH = 128
NCHUNK = EW // CH
SB = 10
NSB = NCHUNK // SB
PAIRS = SB // 2
NP = 10240
RPS = NP // NS


def _fill_2d(ref, rows, width, value):
    def row(i, _):
        def col(j, _):
            ref[i, pl.ds(j * 16, 16)] = jnp.full((16,), value, jnp.float32)
            return 0
        lax.fori_loop(0, width // 16, col, 0)
        return 0
    lax.fori_loop(0, rows, row, 0)


def _zero_own_slice(rows_v, acc, s):
    _fill_2d(rows_v, CH, D, 0.0)
    for k in range(RPS // CH):
        pltpu.sync_copy(rows_v, acc.at[pl.ds(s * RPS + k * CH, CH)])


def _copy_out(acc, out_hbm, c, s):
    pltpu.sync_copy(acc.at[pl.ds(s * RPS, RPS)],
                    out_hbm.at[c, pl.ds(s * RPS, RPS)])


def _sc_agg_body(y_hbm, src_hbm, dst_hbm, out_hbm, src_v, dst_v,
                 rows_a, rows_b, acc, sem_ga, sem_gb, sem_sa, sem_sb):
    c = lax.axis_index("c")
    s = lax.axis_index("s")
    wid = c * NS + s

    _zero_own_slice(rows_a, acc, s)
    plsc.subcore_barrier()

    def gather(buf, sem, j):
        pltpu.async_copy(y_hbm.at[src_v.at[j]], buf, sem)

    def gather_wait(buf, sem, j):
        pltpu.make_async_copy(y_hbm.at[src_v.at[j]], buf, sem).wait()

    def scat(buf, sem, j):
        pltpu.async_copy(buf, acc.at[dst_v.at[j]], sem, add=True)

    def scat_wait(buf, sem, j):
        pltpu.make_async_copy(buf, acc.at[dst_v.at[j]], sem).wait()

    def superchunk(sb, _):
        pltpu.sync_copy(src_hbm.at[wid, sb], src_v)
        pltpu.sync_copy(dst_hbm.at[wid, sb], dst_v)
        gather(rows_a, sem_ga, 0)
        gather(rows_b, sem_gb, 1)

        def pair(p, _):
            j0 = 2 * p
            j1 = j0 + 1
            gather_wait(rows_a, sem_ga, j0)
            scat(rows_a, sem_sa, j0)
            gather_wait(rows_b, sem_gb, j1)
            scat(rows_b, sem_sb, j1)

            @pl.when(p < PAIRS - 1)
            def _():
                scat_wait(rows_a, sem_sa, j0)
                gather(rows_a, sem_ga, j0 + 2)
                scat_wait(rows_b, sem_sb, j1)
                gather(rows_b, sem_gb, j1 + 2)
            return 0
        lax.fori_loop(0, PAIRS, pair, 0)
        scat_wait(rows_a, sem_sa, SB - 2)
        scat_wait(rows_b, sem_sb, SB - 1)
        return 0
    lax.fori_loop(0, NSB, superchunk, 0)

    plsc.subcore_barrier()
    _copy_out(acc, out_hbm, c, s)


def _sc_cnt_body(dst_hbm, out_hbm, dst_v, rows_v, acc, sem):
    c = lax.axis_index("c")
    s = lax.axis_index("s")
    wid = c * NS + s

    _zero_own_slice(rows_v, acc, s)
    plsc.subcore_barrier()

    _fill_2d(rows_v, CH, D, 1.0)

    def superchunk(sb, _):
        pltpu.sync_copy(dst_hbm.at[wid, sb], dst_v)

        def group(g, _):
            for t in range(5):
                pltpu.async_copy(rows_v, acc.at[dst_v.at[g * 5 + t]], sem,
                                 add=True)
            for t in range(5):
                pltpu.make_async_copy(rows_v, acc.at[dst_v.at[g * 5 + t]],
                                      sem).wait()
            return 0
        lax.fori_loop(0, SB // 5, group, 0)
        return 0
    lax.fori_loop(0, NSB, superchunk, 0)

    plsc.subcore_barrier()
    _copy_out(acc, out_hbm, c, s)


@functools.lru_cache(maxsize=None)
def _make_sc_agg():
    mesh = plsc.VectorSubcoreMesh(core_axis_name="c", subcore_axis_name="s",
                                  num_cores=NC, num_subcores=NS)
    return pl.kernel(
        _sc_agg_body,
        out_type=[jax.ShapeDtypeStruct((NC, NP, D), jnp.float32)],
        mesh=mesh,
        scratch_types=[
            pltpu.VMEM((SB, CH), jnp.int32),
            pltpu.VMEM((SB, CH), jnp.int32),
            pltpu.VMEM((CH, D), jnp.float32),
            pltpu.VMEM((CH, D), jnp.float32),
            pltpu.VMEM_SHARED((NP, D), jnp.float32),
            pltpu.SemaphoreType.DMA,
            pltpu.SemaphoreType.DMA,
            pltpu.SemaphoreType.DMA,
            pltpu.SemaphoreType.DMA,
        ],
    )


@functools.lru_cache(maxsize=None)
def _make_sc_cnt():
    mesh = plsc.VectorSubcoreMesh(core_axis_name="c", subcore_axis_name="s",
                                  num_cores=NC, num_subcores=NS)
    return pl.kernel(
        _sc_cnt_body,
        out_type=[jax.ShapeDtypeStruct((NC, NP, D), jnp.float32)],
        mesh=mesh,
        scratch_types=[
            pltpu.VMEM((SB, CH), jnp.int32),
            pltpu.VMEM((CH, D), jnp.float32),
            pltpu.VMEM_SHARED((NP, D), jnp.float32),
            pltpu.SemaphoreType.DMA,
        ],
    )


def _tc_combine_body(relu, padded, rb, acc_ref, cnt_ref, h_ref, wl_ref,
                     b_ref, wr_ref, out_ref):
    a = acc_ref[0] + acc_ref[1]
    cnt = cnt_ref[0, :, 0:1] + cnt_ref[1, :, 0:1]
    inv = 1.0 / jnp.maximum(cnt, 1.0)
    m = a * inv
    out = (jnp.dot(m, wl_ref[...], preferred_element_type=jnp.float32)
           + b_ref[...]
           + jnp.dot(h_ref[...], wr_ref[...],
                     preferred_element_type=jnp.float32))
    if relu:
        out = jnp.maximum(out, 0.0)
    if padded:
        rid = (jax.lax.broadcasted_iota(jnp.int32, (rb, 1), 0)
               + pl.program_id(0) * rb)
        out = jnp.where(rid < N, out, 0.0)
    out_ref[...] = out


@functools.lru_cache(maxsize=None)
def _make_combine(relu, padded):
    rb = 1024 if padded else 1000
    n_out = NP if padded else N
    return pl.pallas_call(
        functools.partial(_tc_combine_body, relu, padded, rb),
        grid=(n_out // rb,),
        in_specs=[
            pl.BlockSpec((NC, rb, D), lambda i: (0, i, 0)),
            pl.BlockSpec((NC, rb, D), lambda i: (0, i, 0)),
            pl.BlockSpec((rb, D), lambda i: (i, 0)),
            pl.BlockSpec((D, D), lambda i: (0, 0)),
            pl.BlockSpec((1, D), lambda i: (0, 0)),
            pl.BlockSpec((D, D), lambda i: (0, 0)),
        ],
        out_specs=pl.BlockSpec((rb, D), lambda i: (i, 0)),
        out_shape=jax.ShapeDtypeStruct((n_out, D), jnp.float32),
    )


def kernel(x, edge_index, W_l0, b_l0, W_r0, W_l1, b_l1, W_r1, W_l2, b_l2,
           W_r2):
    PW = EPAD // NW
    k = jnp.arange(PW, dtype=jnp.int32)[None, :]
    w = jnp.arange(NW, dtype=jnp.int32)[:, None]
    pad_src = jnp.broadcast_to(N + k % (NP - N), (NW, PW))
    pad_dst_agg = (w * 313 + k * 41) % N
    pad_dst_cnt = jnp.broadcast_to(N + k % (NP - N), (NW, PW))
    r_src = edge_index[0].reshape(NW, E // NW)
    r_dst = edge_index[1].reshape(NW, E // NW)
    shp = (NW, NSB, SB, CH)
    src = jnp.concatenate([r_src, pad_src], axis=1).reshape(shp)
    dst = jnp.concatenate([r_dst, pad_dst_agg], axis=1).reshape(shp)
    dst_c = jnp.concatenate([r_dst, pad_dst_cnt], axis=1).reshape(shp)
    x_p = jnp.concatenate([x, jnp.zeros((NP - N, D), jnp.float32)])

    sc_agg, sc_cnt = _make_sc_agg(), _make_sc_cnt()
    combine_mid = _make_combine(True, True)
    combine_last = _make_combine(False, False)

    (cnt2,) = sc_cnt(dst_c)
    (acc2,) = sc_agg(x_p, src, dst)
    h1 = combine_mid(acc2, cnt2, x_p, W_l0, b_l0.reshape(1, D), W_r0)
    (acc2,) = sc_agg(h1, src, dst)
    h2 = combine_mid(acc2, cnt2, h1, W_l1, b_l1.reshape(1, D), W_r1)
    (acc2,) = sc_agg(h2, src, dst)
    return combine_last(acc2, cnt2, h2, W_l2, b_l2.reshape(1, D), W_r2)


# speedup vs baseline: 2.6095x; 1.0236x over previous
import functools

import jax
import jax.numpy as jnp
from jax import lax
from jax.experimental import pallas as pl
from jax.experimental.pallas import tpu as pltpu
from jax.experimental.pallas import tpu_sc as plsc

N = 10000
E = 320000
D = 128
DA = 144

NC = 2
NS = 16
NW = NC * NS
EP = 327680
EPAD = EP - E
EW = EP // NW
NP = 10240
RPS = NP // NS

CH_D, SB_D = 128, 10
CH_A, SB_A = 80, 16


def _fill_2d(ref, rows, width, value):
    def row(i, _):
        def col(j, _):
            ref[i, pl.ds(j * 16, 16)] = jnp.full((16,), value, jnp.float32)
            return 0
        lax.fori_loop(0, width // 16, col, 0)
        return 0
    lax.fori_loop(0, rows, row, 0)


def _sc_agg_body(w, ch, sb, y_hbm, src_hbm, dst_hbm, out_hbm, src_v, dst_v,
                 rows_a, rows_b, acc, sem_ga, sem_gb, sem_sa, sem_sb):
    pairs = sb // 2
    nsb = (EW // ch) // sb
    c = lax.axis_index("c")
    s = lax.axis_index("s")
    wid = c * NS + s

    _fill_2d(rows_a, ch, w, 0.0)
    for k in range(RPS // ch):
        pltpu.sync_copy(rows_a, acc.at[pl.ds(s * RPS + k * ch, ch)])
    plsc.subcore_barrier()

    def gather(buf, sem, j):
        pltpu.async_copy(y_hbm.at[src_v.at[j]], buf, sem)

    def gather_wait(buf, sem, j):
        pltpu.make_async_copy(y_hbm.at[src_v.at[j]], buf, sem).wait()

    def scat(buf, sem, j):
        pltpu.async_copy(buf, acc.at[dst_v.at[j]], sem, add=True)

    def scat_wait(buf, sem, j):
        pltpu.make_async_copy(buf, acc.at[dst_v.at[j]], sem).wait()

    def superchunk(g, _):
        pltpu.sync_copy(src_hbm.at[wid, g], src_v)
        pltpu.sync_copy(dst_hbm.at[wid, g], dst_v)
        gather(rows_a, sem_ga, 0)
        gather(rows_b, sem_gb, 1)

        def pair(p, _):
            j0 = 2 * p
            j1 = j0 + 1
            gather_wait(rows_a, sem_ga, j0)
            scat(rows_a, sem_sa, j0)
            gather_wait(rows_b, sem_gb, j1)
            scat(rows_b, sem_sb, j1)

            @pl.when(p < pairs - 1)
            def _():
                scat_wait(rows_a, sem_sa, j0)
                gather(rows_a, sem_ga, j0 + 2)
                scat_wait(rows_b, sem_sb, j1)
                gather(rows_b, sem_gb, j1 + 2)
            return 0
        lax.fori_loop(0, pairs, pair, 0)
        scat_wait(rows_a, sem_sa, sb - 2)
        scat_wait(rows_b, sem_sb, sb - 1)
        return 0
    lax.fori_loop(0, nsb, superchunk, 0)

    plsc.subcore_barrier()
    pltpu.sync_copy(acc.at[pl.ds(s * RPS, RPS)],
                    out_hbm.at[c, pl.ds(s * RPS, RPS)])


@functools.lru_cache(maxsize=None)
def _make_sc_agg(w, ch, sb):
    mesh = plsc.VectorSubcoreMesh(core_axis_name="c", subcore_axis_name="s",
                                  num_cores=NC, num_subcores=NS)
    return pl.kernel(
        functools.partial(_sc_agg_body, w, ch, sb),
        out_type=[jax.ShapeDtypeStruct((NC, NP, w), jnp.float32)],
        mesh=mesh,
        compiler_params=pltpu.CompilerParams(
            use_tc_tiling_on_sc=(w % 128 == 0)),
        scratch_types=[
            pltpu.VMEM((sb, ch), jnp.int32),
            pltpu.VMEM((sb, ch), jnp.int32),
            pltpu.VMEM((ch, w), jnp.float32),
            pltpu.VMEM((ch, w), jnp.float32),
            pltpu.VMEM_SHARED((NP, w), jnp.float32),
            pltpu.SemaphoreType.DMA,
            pltpu.SemaphoreType.DMA,
            pltpu.SemaphoreType.DMA,
            pltpu.SemaphoreType.DMA,
        ],
    )


def _combine_math(relu, padded, rb, a, cnt, h_ref, wl_ref, b_ref, wr_ref,
                  out_ref):
    inv = 1.0 / jnp.maximum(cnt, 1.0)
    m = a * inv
    out = (jnp.dot(m, wl_ref[...], preferred_element_type=jnp.float32)
           + b_ref[...]
           + jnp.dot(h_ref[...], wr_ref[...],
                     preferred_element_type=jnp.float32))
    if relu:
        out = jnp.maximum(out, 0.0)
    if padded:
        rid = (jax.lax.broadcasted_iota(jnp.int32, (rb, 1), 0)
               + pl.program_id(0) * rb)
        out = jnp.where(rid < N, out, 0.0)
    out_ref[...] = out


def _tc_combine0_body(rb, acc_ref, h_ref, wl_ref, b_ref, wr_ref, out_ref,
                      cnt_out_ref):
    aug = acc_ref[0] + acc_ref[1]
    cnt = aug[:, D:D + 1]
    cnt_out_ref[...] = cnt
    _combine_math(True, True, rb, aug[:, :D], cnt, h_ref, wl_ref, b_ref,
                  wr_ref, out_ref)


def _tc_combine_body(relu, padded, rb, acc_ref, cnt_ref, h_ref, wl_ref,
                     b_ref, wr_ref, out_ref):
    a = acc_ref[0] + acc_ref[1]
    _combine_math(relu, padded, rb, a, cnt_ref[...], h_ref, wl_ref, b_ref,
                  wr_ref, out_ref)


@functools.lru_cache(maxsize=None)
def _make_combine0():
    rb = 1024
    return pl.pallas_call(
        functools.partial(_tc_combine0_body, rb),
        grid=(NP // rb,),
        in_specs=[
            pl.BlockSpec((NC, rb, DA), lambda i: (0, i, 0)),
            pl.BlockSpec((rb, D), lambda i: (i, 0)),
            pl.BlockSpec((D, D), lambda i: (0, 0)),
            pl.BlockSpec((1, D), lambda i: (0, 0)),
            pl.BlockSpec((D, D), lambda i: (0, 0)),
        ],
        out_specs=[
            pl.BlockSpec((rb, D), lambda i: (i, 0)),
            pl.BlockSpec((rb, 1), lambda i: (i, 0)),
        ],
        out_shape=[
            jax.ShapeDtypeStruct((NP, D), jnp.float32),
            jax.ShapeDtypeStruct((NP, 1), jnp.float32),
        ],
    )


@functools.lru_cache(maxsize=None)
def _make_combine(relu, padded):
    rb = 1024 if padded else 1000
    n_out = NP if padded else N
    return pl.pallas_call(
        functools.partial(_tc_combine_body, relu, padded, rb),
        grid=(n_out // rb,),
        in_specs=[
            pl.BlockSpec((NC, rb, D), lambda i: (0, i, 0)),
            pl.BlockSpec((rb, 1), lambda i: (i, 0)),
            pl.BlockSpec((rb, D), lambda i: (i, 0)),
            pl.BlockSpec((D, D), lambda i: (0, 0)),
            pl.BlockSpec((1, D), lambda i: (0, 0)),
            pl.BlockSpec((D, D), lambda i: (0, 0)),
        ],
        out_specs=pl.BlockSpec((rb, D), lambda i: (i, 0)),
        out_shape=jax.ShapeDtypeStruct((n_out, D), jnp.float32),
    )


def kernel(x, edge_index, W_l0, b_l0, W_r0, W_l1, b_l1, W_r1, W_l2, b_l2,
           W_r2):
    PW = EPAD // NW
    k = jnp.arange(PW, dtype=jnp.int32)[None, :]
    w = jnp.arange(NW, dtype=jnp.int32)[:, None]
    pad_src = jnp.broadcast_to(N + k % (NP - N), (NW, PW))
    pad_dst = (w * 313 + k * 41) % N
    r_src = edge_index[0].reshape(NW, E // NW)
    r_dst = edge_index[1].reshape(NW, E // NW)
    src_f = jnp.concatenate([r_src, pad_src], axis=1)
    dst_f = jnp.concatenate([r_dst, pad_dst], axis=1)

    def chunked(a, ch, sb):
        return a.reshape(NW, (EW // ch) // sb, sb, ch)

    src_a, dst_a = chunked(src_f, CH_A, SB_A), chunked(dst_f, CH_A, SB_A)
    src_d, dst_d = chunked(src_f, CH_D, SB_D), chunked(dst_f, CH_D, SB_D)

    ones_col = jnp.concatenate([jnp.ones((N, 1), jnp.float32),
                                jnp.zeros((NP - N, 1), jnp.float32)])
    x_p = jnp.concatenate([x, jnp.zeros((NP - N, D), jnp.float32)])
    x_aug = jnp.concatenate(
        [x_p, ones_col, jnp.zeros((NP, DA - D - 1), jnp.float32)], axis=1)

    agg_a = _make_sc_agg(DA, CH_A, SB_A)
    agg_d = _make_sc_agg(D, CH_D, SB_D)
    combine0 = _make_combine0()
    combine_mid = _make_combine(True, True)
    combine_last = _make_combine(False, False)

    (acc2,) = agg_a(x_aug, src_a, dst_a)
    h1, cnt = combine0(acc2, x_p, W_l0, b_l0.reshape(1, D), W_r0)
    (acc2,) = agg_d(h1, src_d, dst_d)
    h2 = combine_mid(acc2, cnt, h1, W_l1, b_l1.reshape(1, D), W_r1)
    (acc2,) = agg_d(h2, src_d, dst_d)
    return combine_last(acc2, cnt, h2, W_l2, b_l2.reshape(1, D), W_r2)


# speedup vs baseline: 3.1308x; 1.1998x over previous
import functools

import jax
import jax.numpy as jnp
from jax import lax
from jax.experimental import pallas as pl
from jax.experimental.pallas import tpu as pltpu
from jax.experimental.pallas import tpu_sc as plsc

N = 10000
E = 320000
D = 128
DA = 160

NC = 2
NS = 16
NW = NC * NS
EP = 327680
EPAD = EP - E
EW = EP // NW
NP = 10240
RPS = NP // NS

CH_D, SB_D = 128, 10
CH_A, SB_A = 128, 10


def _fill_2d(ref, rows, width, value):
    def row(i, _):
        def col(j, _):
            ref[i, pl.ds(j * 32, 32)] = jnp.full((32,), value, jnp.bfloat16)
            return 0
        lax.fori_loop(0, width // 32, col, 0)
        return 0
    lax.fori_loop(0, rows, row, 0)


def _sc_agg_body(w, ch, sb, y_hbm, src_hbm, dst_hbm, out_hbm, src_v, dst_v,
                 rows_a, rows_b, acc, sem_ga, sem_gb, sem_sa, sem_sb):
    pairs = sb // 2
    nsb = (EW // ch) // sb
    c = lax.axis_index("c")
    s = lax.axis_index("s")
    wid = c * NS + s

    _fill_2d(rows_a, ch, w, 0.0)
    for k in range(RPS // ch):
        pltpu.sync_copy(rows_a, acc.at[pl.ds(s * RPS + k * ch, ch)])
    plsc.subcore_barrier()

    def gather(buf, sem, j):
        pltpu.async_copy(y_hbm.at[src_v.at[j]], buf, sem)

    def gather_wait(buf, sem, j):
        pltpu.make_async_copy(y_hbm.at[src_v.at[j]], buf, sem).wait()

    def scat(buf, sem, j):
        pltpu.async_copy(buf, acc.at[dst_v.at[j]], sem, add=True)

    def scat_wait(buf, sem, j):
        pltpu.make_async_copy(buf, acc.at[dst_v.at[j]], sem).wait()

    def superchunk(g, _):
        pltpu.sync_copy(src_hbm.at[wid, g], src_v)
        pltpu.sync_copy(dst_hbm.at[wid, g], dst_v)
        gather(rows_a, sem_ga, 0)
        gather(rows_b, sem_gb, 1)

        def pair(p, _):
            j0 = 2 * p
            j1 = j0 + 1
            gather_wait(rows_a, sem_ga, j0)
            scat(rows_a, sem_sa, j0)
            gather_wait(rows_b, sem_gb, j1)
            scat(rows_b, sem_sb, j1)

            @pl.when(p < pairs - 1)
            def _():
                scat_wait(rows_a, sem_sa, j0)
                gather(rows_a, sem_ga, j0 + 2)
                scat_wait(rows_b, sem_sb, j1)
                gather(rows_b, sem_gb, j1 + 2)
            return 0
        lax.fori_loop(0, pairs, pair, 0)
        scat_wait(rows_a, sem_sa, sb - 2)
        scat_wait(rows_b, sem_sb, sb - 1)
        return 0
    lax.fori_loop(0, nsb, superchunk, 0)

    plsc.subcore_barrier()
    pltpu.sync_copy(acc.at[pl.ds(s * RPS, RPS)],
                    out_hbm.at[c, pl.ds(s * RPS, RPS)])


@functools.lru_cache(maxsize=None)
def _make_sc_agg(w, ch, sb):
    mesh = plsc.VectorSubcoreMesh(core_axis_name="c", subcore_axis_name="s",
                                  num_cores=NC, num_subcores=NS)
    return pl.kernel(
        functools.partial(_sc_agg_body, w, ch, sb),
        out_type=[jax.ShapeDtypeStruct((NC, NP, w), jnp.bfloat16)],
        mesh=mesh,
        compiler_params=pltpu.CompilerParams(use_tc_tiling_on_sc=False),
        scratch_types=[
            pltpu.VMEM((sb, ch), jnp.int32),
            pltpu.VMEM((sb, ch), jnp.int32),
            pltpu.VMEM((ch, w), jnp.bfloat16),
            pltpu.VMEM((ch, w), jnp.bfloat16),
            pltpu.VMEM_SHARED((NP, w), jnp.bfloat16),
            pltpu.SemaphoreType.DMA,
            pltpu.SemaphoreType.DMA,
            pltpu.SemaphoreType.DMA,
            pltpu.SemaphoreType.DMA,
        ],
    )


def _combine_math(relu, padded, rb, a, cnt, h_ref, wl_ref, b_ref, wr_ref):
    inv = 1.0 / jnp.maximum(cnt, 1.0)
    m = a * inv
    out = (jnp.dot(m, wl_ref[...], preferred_element_type=jnp.float32)
           + b_ref[...]
           + jnp.dot(h_ref[...], wr_ref[...],
                     preferred_element_type=jnp.float32))
    if relu:
        out = jnp.maximum(out, 0.0)
    if padded:
        rid = (jax.lax.broadcasted_iota(jnp.int32, (rb, 1), 0)
               + pl.program_id(0) * rb)
        out = jnp.where(rid < N, out, 0.0)
    return out


def _tc_combine0_body(rb, acc_ref, h_ref, wl_ref, b_ref, wr_ref, out_ref,
                      outb_ref, cnt_out_ref):
    aug = (acc_ref[0].astype(jnp.float32)
           + acc_ref[1].astype(jnp.float32))
    cnt = aug[:, D:D + 1]
    cnt_out_ref[...] = cnt
    out = _combine_math(True, True, rb, aug[:, :D], cnt, h_ref, wl_ref,
                        b_ref, wr_ref)
    out_ref[...] = out
    outb_ref[...] = out.astype(jnp.bfloat16)


def _tc_combine_body(relu, padded, rb, acc_ref, cnt_ref, h_ref, wl_ref,
                     b_ref, wr_ref, out_ref, outb_ref):
    a = (acc_ref[0].astype(jnp.float32)
         + acc_ref[1].astype(jnp.float32))
    out = _combine_math(relu, padded, rb, a, cnt_ref[...], h_ref, wl_ref,
                        b_ref, wr_ref)
    out_ref[...] = out
    if outb_ref is not None:
        outb_ref[...] = out.astype(jnp.bfloat16)


@functools.lru_cache(maxsize=None)
def _make_combine0():
    rb = 1024
    return pl.pallas_call(
        functools.partial(_tc_combine0_body, rb),
        grid=(NP // rb,),
        in_specs=[
            pl.BlockSpec((NC, rb, DA), lambda i: (0, i, 0)),
            pl.BlockSpec((rb, D), lambda i: (i, 0)),
            pl.BlockSpec((D, D), lambda i: (0, 0)),
            pl.BlockSpec((1, D), lambda i: (0, 0)),
            pl.BlockSpec((D, D), lambda i: (0, 0)),
        ],
        out_specs=[
            pl.BlockSpec((rb, D), lambda i: (i, 0)),
            pl.BlockSpec((rb, D), lambda i: (i, 0)),
            pl.BlockSpec((rb, 1), lambda i: (i, 0)),
        ],
        out_shape=[
            jax.ShapeDtypeStruct((NP, D), jnp.float32),
            jax.ShapeDtypeStruct((NP, D), jnp.bfloat16),
            jax.ShapeDtypeStruct((NP, 1), jnp.float32),
        ],
    )


@functools.lru_cache(maxsize=None)
def _make_combine(relu, padded):
    rb = 1024 if padded else 1000
    n_out = NP if padded else N
    out_specs = [pl.BlockSpec((rb, D), lambda i: (i, 0))]
    out_shape = [jax.ShapeDtypeStruct((n_out, D), jnp.float32)]
    if padded:
        out_specs.append(pl.BlockSpec((rb, D), lambda i: (i, 0)))
        out_shape.append(jax.ShapeDtypeStruct((n_out, D), jnp.bfloat16))
        body = functools.partial(_tc_combine_body, relu, padded, rb)
    else:
        def body(*refs):
            _tc_combine_body(relu, padded, rb, *refs, None)
    return pl.pallas_call(
        body,
        grid=(n_out // rb,),
        in_specs=[
            pl.BlockSpec((NC, rb, D), lambda i: (0, i, 0)),
            pl.BlockSpec((rb, 1), lambda i: (i, 0)),
            pl.BlockSpec((rb, D), lambda i: (i, 0)),
            pl.BlockSpec((D, D), lambda i: (0, 0)),
            pl.BlockSpec((1, D), lambda i: (0, 0)),
            pl.BlockSpec((D, D), lambda i: (0, 0)),
        ],
        out_specs=out_specs,
        out_shape=out_shape,
    )


def kernel(x, edge_index, W_l0, b_l0, W_r0, W_l1, b_l1, W_r1, W_l2, b_l2,
           W_r2):
    PW = EPAD // NW
    k = jnp.arange(PW, dtype=jnp.int32)[None, :]
    w = jnp.arange(NW, dtype=jnp.int32)[:, None]
    pad_src = jnp.broadcast_to(N + k % (NP - N), (NW, PW))
    pad_dst = (w * 313 + k * 41) % N
    r_src = edge_index[0].reshape(NW, E // NW)
    r_dst = edge_index[1].reshape(NW, E // NW)
    src_f = jnp.concatenate([r_src, pad_src], axis=1)
    dst_f = jnp.concatenate([r_dst, pad_dst], axis=1)

    def chunked(a, ch, sb):
        return a.reshape(NW, (EW // ch) // sb, sb, ch)

    src_a, dst_a = chunked(src_f, CH_A, SB_A), chunked(dst_f, CH_A, SB_A)
    src_d, dst_d = chunked(src_f, CH_D, SB_D), chunked(dst_f, CH_D, SB_D)

    ones_col = jnp.concatenate([jnp.ones((N, 1), jnp.bfloat16),
                                jnp.zeros((NP - N, 1), jnp.bfloat16)])
    x_p = jnp.concatenate([x, jnp.zeros((NP - N, D), jnp.float32)])
    x_aug = jnp.concatenate(
        [x_p.astype(jnp.bfloat16), ones_col,
         jnp.zeros((NP, DA - D - 1), jnp.bfloat16)], axis=1)

    agg_a = _make_sc_agg(DA, CH_A, SB_A)
    agg_d = _make_sc_agg(D, CH_D, SB_D)
    combine0 = _make_combine0()
    combine_mid = _make_combine(True, True)
    combine_last = _make_combine(False, False)

    (acc2,) = agg_a(x_aug, src_a, dst_a)
    h1, h1b, cnt = combine0(acc2, x_p, W_l0, b_l0.reshape(1, D), W_r0)
    (acc2,) = agg_d(h1b, src_d, dst_d)
    h2, h2b = combine_mid(acc2, cnt, h1, W_l1, b_l1.reshape(1, D), W_r1)
    (acc2,) = agg_d(h2b, src_d, dst_d)
    (out,) = combine_last(acc2, cnt, h2, W_l2, b_l2.reshape(1, D), W_r2)
    return out
